# baseline (speedup 1.0000x reference)
import jax
import jax.numpy as jnp
from jax import lax
from jax.experimental import pallas as pl
from jax.experimental.pallas import tpu as pltpu
from jax.experimental.pallas import tpu_sc as plsc

_D = 64
_BATCH = 16384
_L = 16
_NW = 32
_BPW = _BATCH // _NW
_NG = _BPW // _L
_RING = 4
_SLAB = _D * 128


def _extract_body(usid_hbm, upos_hbm, isid_hbm, ipos_hbm, ut_hbm, it_hbm,
                  ue_hbm, ie_hbm,
                  sid_v, pos_v, slabq_v, startq_v, slab_v, stage_v,
                  fsem, osem):
    wid = lax.axis_index("c") * 16 + lax.axis_index("s")
    base = wid * _BPW
    cvec = lax.iota(jnp.int32, _L)

    def phase(sid_hbm, pos_hbm, tab_hbm, dst_hbm):
        pltpu.sync_copy(sid_hbm.at[pl.ds(base, _BPW)], sid_v.at[pl.ds(0, _BPW)])
        pltpu.sync_copy(pos_hbm.at[pl.ds(base, _BPW)], pos_v.at[pl.ds(0, _BPW)])

        def scan(v, carry):
            off, last = carry
            idv = sid_v[pl.ds(v * _L, _L)]
            tv = idv >> 7
            shifted = tv[jnp.maximum(cvec - 1, 0)]
            rolled = jnp.where(cvec == 0, last, shifted)
            m = tv != rolled
            plsc.store_compressed(slabq_v.at[pl.ds(off, _L)], tv, mask=m)
            plsc.store_compressed(startq_v.at[pl.ds(off, _L)],
                                  v * _L + cvec, mask=m)
            cnt = plsc.all_reduce_population_count(m)[0]
            return off + cnt, tv[_L - 1]

        nslab, _last = lax.fori_loop(
            0, _NG, scan, (jnp.int32(0), jnp.int32(-1)))
        startq_v[pl.ds(nslab, _L)] = jnp.full((_L,), 1, jnp.int32) * _BPW

        def fire(s):
            t = slabq_v[pl.ds(s, _L)][0]
            off = pl.multiple_of(t * 128, 128)
            slot = lax.rem(s, _RING)
            pltpu.async_copy(tab_hbm.at[:, pl.ds(off, 128)],
                             slab_v.at[slot], fsem)

        def prime(s, carry):
            fire(s)
            return carry

        lax.fori_loop(0, jnp.minimum(nslab, _RING - 1), prime, 0)

        def do_slab(s, nfired):
            pltpu.make_async_copy(
                tab_hbm.at[:, pl.ds(0, 128)], slab_v.at[0], fsem).wait()
            slot = lax.rem(s, _RING)
            b0 = startq_v[pl.ds(s, _L)][0]
            b1 = startq_v[pl.ds(s + 1, _L)][0]

            def item(b, nfired):
                sslot = lax.rem(b, _L)

                @pl.when(jnp.logical_and(sslot == 0, nfired > 0))
                def _drain():
                    def d(i, c):
                        pltpu.make_async_copy(
                            ue_hbm.at[pl.ds(0, _D)],
                            stage_v.at[pl.ds(0, _D)], osem).wait()
                        return c
                    lax.fori_loop(0, nfired, d, 0)

                nfired = jnp.where(sslot == 0, 0, nfired)
                idw = sid_v[pl.ds(b, _L)][0]
                pos = pos_v[pl.ds(b, _L)][0]
                lvv = jnp.full((_L,), 1, jnp.int32) * (idw & 127)
                for k in range(4):
                    col = plsc.load_gather(
                        slab_v, [jnp.full((_L,), 1, jnp.int32) * slot,
                                 cvec + k * _L, lvv])
                    stage_v[pl.ds(sslot * _D + k * _L, _L)] = col
                pltpu.async_copy(stage_v.at[pl.ds(sslot * _D, _D)],
                                 dst_hbm.at[pl.ds(pos * _D, _D)], osem)
                return nfired + 1

            nfired = lax.fori_loop(b0, b1, item, nfired)

            @pl.when(s + _RING - 1 < nslab)
            def _ahead():
                fire(s + _RING - 1)

            return nfired

        nfired = lax.fori_loop(0, nslab, do_slab, jnp.int32(0))

        def dtail(i, c):
            pltpu.make_async_copy(ue_hbm.at[pl.ds(0, _D)],
                                  stage_v.at[pl.ds(0, _D)], osem).wait()
            return c

        lax.fori_loop(0, nfired, dtail, 0)

    phase(usid_hbm, upos_hbm, ut_hbm, ue_hbm)
    phase(isid_hbm, ipos_hbm, it_hbm, ie_hbm)


def _dot_body(ue_hbm, ie_hbm, out_hbm, uv, iv, ov):
    wid = lax.axis_index("c") * 16 + lax.axis_index("s")
    pltpu.sync_copy(ue_hbm.at[pl.ds(wid * _BPW * _D, _BPW * _D)], uv)
    pltpu.sync_copy(ie_hbm.at[pl.ds(wid * _BPW * _D, _BPW * _D)], iv)
    iota = lax.iota(jnp.int32, _L)

    def group(g, carry):
        rb = (g * _L + iota) * _D
        accs = [jnp.zeros((_L,), jnp.float32) for _ in range(4)]
        for f in range(_D):
            u = plsc.load_gather(uv, [rb + f])
            i2 = plsc.load_gather(iv, [rb + f])
            accs[f % 4] = accs[f % 4] + u * i2
        acc = (accs[0] + accs[1]) + (accs[2] + accs[3])
        ov[pl.ds(g * _L, _L)] = 1.0 / (1.0 + jnp.exp(-acc))
        return carry

    lax.fori_loop(0, _NG, group, 0)
    pltpu.sync_copy(ov, out_hbm.at[pl.ds(wid * _BPW, _BPW)])


@jax.jit
def kernel(user, item_i, embed_user_weight, embed_item_weight):
    u32 = user.astype(jnp.int32)
    i32 = item_i.astype(jnp.int32)
    posa = lax.iota(jnp.int32, _BATCH)
    usid, upos = lax.sort_key_val(u32, posa)
    isid, ipos = lax.sort_key_val(i32, posa)
    ut = jnp.transpose(embed_user_weight)
    it = jnp.transpose(embed_item_weight)

    mesh = plsc.VectorSubcoreMesh(core_axis_name="c", subcore_axis_name="s")
    params = pltpu.CompilerParams(
        needs_layout_passes=False, disable_bounds_checks=True)

    extract = pl.kernel(
        _extract_body,
        out_type=(jax.ShapeDtypeStruct((_BATCH * _D,), jnp.float32),
                  jax.ShapeDtypeStruct((_BATCH * _D,), jnp.float32)),
        mesh=mesh,
        compiler_params=params,
        scratch_types=[
            pltpu.VMEM((_BPW + _L,), jnp.int32),
            pltpu.VMEM((_BPW + _L,), jnp.int32),
            pltpu.VMEM((_BPW + 3 * _L,), jnp.int32),
            pltpu.VMEM((_BPW + 3 * _L,), jnp.int32),
            pltpu.VMEM((_RING, _D, 128), jnp.float32),
            pltpu.VMEM((_L * _D,), jnp.float32),
            pltpu.SemaphoreType.DMA,
            pltpu.SemaphoreType.DMA,
        ],
    )
    ue, ie = extract(usid, upos, isid, ipos, ut, it)

    dot = pl.kernel(
        _dot_body,
        out_type=jax.ShapeDtypeStruct((_BATCH,), jnp.float32),
        mesh=mesh,
        compiler_params=params,
        scratch_types=[
            pltpu.VMEM((_BPW * _D,), jnp.float32),
            pltpu.VMEM((_BPW * _D,), jnp.float32),
            pltpu.VMEM((_BPW,), jnp.float32),
        ],
    )
    return dot(ue, ie)


# speedup vs baseline: 1.1534x; 1.1534x over previous
import jax
import jax.numpy as jnp
from jax import lax
from jax.experimental import pallas as pl
from jax.experimental.pallas import tpu as pltpu
from jax.experimental.pallas import tpu_sc as plsc

_D = 64
_BATCH = 16384
_L = 16
_NW = 32
_BPW = _BATCH // _NW
_NG = _BPW // _L
_RING = 4
_SLAB = _D * 128


def _extract_body(usid_hbm, upos_hbm, isid_hbm, ipos_hbm, ut_hbm, it_hbm,
                  ue_hbm, ie_hbm,
                  sid_v, pos_v, slabq_v, startq_v, slab_v, stage_v,
                  fsem, osem):
    wid = lax.axis_index("c") * 16 + lax.axis_index("s")
    base = wid * _BPW
    cvec = lax.iota(jnp.int32, _L)

    def phase(sid_hbm, pos_hbm, tab_hbm, dst_hbm):
        pltpu.sync_copy(sid_hbm.at[pl.ds(base, _BPW)], sid_v.at[pl.ds(0, _BPW)])
        pltpu.sync_copy(pos_hbm.at[pl.ds(base, _BPW)], pos_v.at[pl.ds(0, _BPW)])

        def scan(v, carry):
            off, last = carry
            idv = sid_v[pl.ds(v * _L, _L)]
            tv = idv >> 7
            shifted = tv[jnp.maximum(cvec - 1, 0)]
            rolled = jnp.where(cvec == 0, last, shifted)
            m = tv != rolled
            plsc.store_compressed(slabq_v.at[pl.ds(off, _L)], tv, mask=m)
            plsc.store_compressed(startq_v.at[pl.ds(off, _L)],
                                  v * _L + cvec, mask=m)
            cnt = plsc.all_reduce_population_count(m)[0]
            return off + cnt, tv[_L - 1]

        nslab, _last = lax.fori_loop(
            0, _NG, scan, (jnp.int32(0), jnp.int32(-1)))
        startq_v[pl.ds(nslab, _L)] = jnp.full((_L,), 1, jnp.int32) * _BPW

        def fire(s):
            t = slabq_v[pl.ds(s, _L)][0]
            off = pl.multiple_of(t * 128, 128)
            slot = lax.rem(s, _RING)
            pltpu.async_copy(tab_hbm.at[:, pl.ds(off, 128)],
                             slab_v.at[slot], fsem)

        def prime(s, carry):
            fire(s)
            return carry

        lax.fori_loop(0, jnp.minimum(nslab, _RING - 1), prime, 0)

        def do_slab(s, nfired):
            pltpu.make_async_copy(
                tab_hbm.at[:, pl.ds(0, 128)], slab_v.at[0], fsem).wait()
            slot = lax.rem(s, _RING)
            b0 = startq_v[pl.ds(s, _L)][0]
            b1 = startq_v[pl.ds(s + 1, _L)][0]

            def item(b, nfired):
                sslot = lax.rem(b, _L)

                @pl.when(jnp.logical_and(sslot == 0, nfired > 0))
                def _drain():
                    def d(i, c):
                        pltpu.make_async_copy(
                            ue_hbm.at[pl.ds(0, _D)],
                            stage_v.at[pl.ds(0, _D)], osem).wait()
                        return c
                    lax.fori_loop(0, nfired, d, 0)

                nfired = jnp.where(sslot == 0, 0, nfired)
                idw = sid_v[pl.ds(b, _L)][0]
                pos = pos_v[pl.ds(b, _L)][0]
                lvv = jnp.full((_L,), 1, jnp.int32) * (idw & 127)
                for k in range(4):
                    col = plsc.load_gather(
                        slab_v, [jnp.full((_L,), 1, jnp.int32) * slot,
                                 cvec + k * _L, lvv])
                    stage_v[pl.ds(sslot * _D + k * _L, _L)] = col
                pltpu.async_copy(stage_v.at[pl.ds(sslot * _D, _D)],
                                 dst_hbm.at[pl.ds(pos * _D, _D)], osem)
                return nfired + 1

            nfired = lax.fori_loop(b0, b1, item, nfired)

            @pl.when(s + _RING - 1 < nslab)
            def _ahead():
                fire(s + _RING - 1)

            return nfired

        nfired = lax.fori_loop(0, nslab, do_slab, jnp.int32(0))

        def dtail(i, c):
            pltpu.make_async_copy(ue_hbm.at[pl.ds(0, _D)],
                                  stage_v.at[pl.ds(0, _D)], osem).wait()
            return c

        lax.fori_loop(0, nfired, dtail, 0)

    phase(usid_hbm, upos_hbm, ut_hbm, ue_hbm)
    phase(isid_hbm, ipos_hbm, it_hbm, ie_hbm)


def _dot_body(ue_hbm, ie_hbm, out_hbm, uv, iv, ov):
    wid = lax.axis_index("c") * 16 + lax.axis_index("s")
    pltpu.sync_copy(ue_hbm.at[pl.ds(wid * _BPW * _D, _BPW * _D)], uv)
    pltpu.sync_copy(ie_hbm.at[pl.ds(wid * _BPW * _D, _BPW * _D)], iv)
    iota = lax.iota(jnp.int32, _L)

    def group(g, carry):
        acc = jnp.zeros((_L,), jnp.float32)
        for j in range(_L):
            bp = (g * _L + j) * _D
            p = jnp.zeros((_L,), jnp.float32)
            for k in range(4):
                u = uv[pl.ds(bp + k * _L, _L)]
                i2 = iv[pl.ds(bp + k * _L, _L)]
                p = p + u * i2
            s = jnp.sum(p)
            acc = jnp.where(iota == j, s, acc)
        ov[pl.ds(g * _L, _L)] = 1.0 / (1.0 + jnp.exp(-acc))
        return carry

    lax.fori_loop(0, _NG, group, 0)
    pltpu.sync_copy(ov, out_hbm.at[pl.ds(wid * _BPW, _BPW)])


@jax.jit
def kernel(user, item_i, embed_user_weight, embed_item_weight):
    u32 = user.astype(jnp.int32)
    i32 = item_i.astype(jnp.int32)
    posa = lax.iota(jnp.int32, _BATCH)
    usid, upos = lax.sort_key_val(u32, posa)
    isid, ipos = lax.sort_key_val(i32, posa)
    ut = jnp.transpose(embed_user_weight)
    it = jnp.transpose(embed_item_weight)

    mesh = plsc.VectorSubcoreMesh(core_axis_name="c", subcore_axis_name="s")
    params = pltpu.CompilerParams(
        needs_layout_passes=False, disable_bounds_checks=True)

    extract = pl.kernel(
        _extract_body,
        out_type=(jax.ShapeDtypeStruct((_BATCH * _D,), jnp.float32),
                  jax.ShapeDtypeStruct((_BATCH * _D,), jnp.float32)),
        mesh=mesh,
        compiler_params=params,
        scratch_types=[
            pltpu.VMEM((_BPW + _L,), jnp.int32),
            pltpu.VMEM((_BPW + _L,), jnp.int32),
            pltpu.VMEM((_BPW + 3 * _L,), jnp.int32),
            pltpu.VMEM((_BPW + 3 * _L,), jnp.int32),
            pltpu.VMEM((_RING, _D, 128), jnp.float32),
            pltpu.VMEM((_L * _D,), jnp.float32),
            pltpu.SemaphoreType.DMA,
            pltpu.SemaphoreType.DMA,
        ],
    )
    ue, ie = extract(usid, upos, isid, ipos, ut, it)

    dot = pl.kernel(
        _dot_body,
        out_type=jax.ShapeDtypeStruct((_BATCH,), jnp.float32),
        mesh=mesh,
        compiler_params=params,
        scratch_types=[
            pltpu.VMEM((_BPW * _D,), jnp.float32),
            pltpu.VMEM((_BPW * _D,), jnp.float32),
            pltpu.VMEM((_BPW,), jnp.float32),
        ],
    )
    return dot(ue, ie)


# speedup vs baseline: 1.3423x; 1.1638x over previous
import jax
import jax.numpy as jnp
from jax import lax
from jax.experimental import pallas as pl
from jax.experimental.pallas import tpu as pltpu
from jax.experimental.pallas import tpu_sc as plsc

_D = 64
_BATCH = 16384
_L = 16
_NW = 32
_BPW = _BATCH // _NW
_NG = _BPW // _L
_RING = 8
_SLAB = _D * 128


def _extract_body(usid_hbm, upos_hbm, isid_hbm, ipos_hbm, ut_hbm, it_hbm,
                  ue_hbm, ie_hbm,
                  sid_v, pos_v, slabq_v, startq_v, slab_v, stage_v,
                  fsem, osem):
    wid = lax.axis_index("c") * 16 + lax.axis_index("s")
    base = wid * _BPW
    cvec = lax.iota(jnp.int32, _L)

    def phase(sid_hbm, pos_hbm, tab_hbm, dst_hbm):
        pltpu.sync_copy(sid_hbm.at[pl.ds(base, _BPW)], sid_v.at[pl.ds(0, _BPW)])
        pltpu.sync_copy(pos_hbm.at[pl.ds(base, _BPW)], pos_v.at[pl.ds(0, _BPW)])

        def scan(v, carry):
            off, last = carry
            idv = sid_v[pl.ds(v * _L, _L)]
            tv = idv >> 7
            shifted = tv[jnp.maximum(cvec - 1, 0)]
            rolled = jnp.where(cvec == 0, last, shifted)
            m = tv != rolled
            plsc.store_compressed(slabq_v.at[pl.ds(off, _L)], tv, mask=m)
            plsc.store_compressed(startq_v.at[pl.ds(off, _L)],
                                  v * _L + cvec, mask=m)
            cnt = plsc.all_reduce_population_count(m)[0]
            return off + cnt, tv[_L - 1]

        nslab, _last = lax.fori_loop(
            0, _NG, scan, (jnp.int32(0), jnp.int32(-1)))
        startq_v[pl.ds(nslab, _L)] = jnp.full((_L,), 1, jnp.int32) * _BPW

        def fire(s):
            t = slabq_v[pl.ds(s, _L)][0]
            off = pl.multiple_of(t * 128, 128)
            slot = lax.rem(s, _RING)
            pltpu.async_copy(tab_hbm.at[:, pl.ds(off, 128)],
                             slab_v.at[slot], fsem)

        def prime(s, carry):
            fire(s)
            return carry

        lax.fori_loop(0, jnp.minimum(nslab, _RING - 1), prime, 0)

        def do_slab(s, nfired):
            pltpu.make_async_copy(
                tab_hbm.at[:, pl.ds(0, 128)], slab_v.at[0], fsem).wait()
            slot = lax.rem(s, _RING)

            @pl.when(s + _RING - 1 < nslab)
            def _ahead():
                fire(s + _RING - 1)

            b0 = startq_v[pl.ds(s, _L)][0]
            b1 = startq_v[pl.ds(s + 1, _L)][0]

            def item(b, nfired):
                sslot = lax.rem(b, _L)

                @pl.when(jnp.logical_and(sslot == 0, nfired > 0))
                def _drain():
                    def d(i, c):
                        pltpu.make_async_copy(
                            ue_hbm.at[pl.ds(0, _D)],
                            stage_v.at[pl.ds(0, _D)], osem).wait()
                        return c
                    lax.fori_loop(0, nfired, d, 0)

                nfired = jnp.where(sslot == 0, 0, nfired)
                idw = sid_v[pl.ds(b, _L)][0]
                pos = pos_v[pl.ds(b, _L)][0]
                lvv = jnp.full((_L,), 1, jnp.int32) * (idw & 127)
                for k in range(4):
                    col = plsc.load_gather(
                        slab_v, [jnp.full((_L,), 1, jnp.int32) * slot,
                                 cvec + k * _L, lvv])
                    stage_v[pl.ds(sslot * _D + k * _L, _L)] = col
                pltpu.async_copy(stage_v.at[pl.ds(sslot * _D, _D)],
                                 dst_hbm.at[pl.ds(pos * _D, _D)], osem)
                return nfired + 1

            nfired = lax.fori_loop(b0, b1, item, nfired)
            return nfired

        nfired = lax.fori_loop(0, nslab, do_slab, jnp.int32(0))

        def dtail(i, c):
            pltpu.make_async_copy(ue_hbm.at[pl.ds(0, _D)],
                                  stage_v.at[pl.ds(0, _D)], osem).wait()
            return c

        lax.fori_loop(0, nfired, dtail, 0)

    phase(usid_hbm, upos_hbm, ut_hbm, ue_hbm)
    phase(isid_hbm, ipos_hbm, it_hbm, ie_hbm)


def _dot_body(ue_hbm, ie_hbm, out_hbm, uv, iv, ov):
    wid = lax.axis_index("c") * 16 + lax.axis_index("s")
    pltpu.sync_copy(ue_hbm.at[pl.ds(wid * _BPW * _D, _BPW * _D)], uv)
    pltpu.sync_copy(ie_hbm.at[pl.ds(wid * _BPW * _D, _BPW * _D)], iv)
    iota = lax.iota(jnp.int32, _L)

    def group(g, carry):
        acc = jnp.zeros((_L,), jnp.float32)
        for j in range(_L):
            bp = (g * _L + j) * _D
            p = jnp.zeros((_L,), jnp.float32)
            for k in range(4):
                u = uv[pl.ds(bp + k * _L, _L)]
                i2 = iv[pl.ds(bp + k * _L, _L)]
                p = p + u * i2
            s = jnp.sum(p)
            acc = jnp.where(iota == j, s, acc)
        ov[pl.ds(g * _L, _L)] = 1.0 / (1.0 + jnp.exp(-acc))
        return carry

    lax.fori_loop(0, _NG, group, 0)
    pltpu.sync_copy(ov, out_hbm.at[pl.ds(wid * _BPW, _BPW)])


@jax.jit
def kernel(user, item_i, embed_user_weight, embed_item_weight):
    u32 = user.astype(jnp.int32)
    i32 = item_i.astype(jnp.int32)
    posa = lax.iota(jnp.int32, _BATCH)
    usid, upos = lax.sort_key_val(u32, posa)
    isid, ipos = lax.sort_key_val(i32, posa)
    ut = jnp.transpose(embed_user_weight)
    it = jnp.transpose(embed_item_weight)

    mesh = plsc.VectorSubcoreMesh(core_axis_name="c", subcore_axis_name="s")
    params = pltpu.CompilerParams(
        needs_layout_passes=False, disable_bounds_checks=True)

    extract = pl.kernel(
        _extract_body,
        out_type=(jax.ShapeDtypeStruct((_BATCH * _D,), jnp.float32),
                  jax.ShapeDtypeStruct((_BATCH * _D,), jnp.float32)),
        mesh=mesh,
        compiler_params=params,
        scratch_types=[
            pltpu.VMEM((_BPW + _L,), jnp.int32),
            pltpu.VMEM((_BPW + _L,), jnp.int32),
            pltpu.VMEM((_BPW + 3 * _L,), jnp.int32),
            pltpu.VMEM((_BPW + 3 * _L,), jnp.int32),
            pltpu.VMEM((_RING, _D, 128), jnp.float32),
            pltpu.VMEM((_L * _D,), jnp.float32),
            pltpu.SemaphoreType.DMA,
            pltpu.SemaphoreType.DMA,
        ],
    )
    ue, ie = extract(usid, upos, isid, ipos, ut, it)

    dot = pl.kernel(
        _dot_body,
        out_type=jax.ShapeDtypeStruct((_BATCH,), jnp.float32),
        mesh=mesh,
        compiler_params=params,
        scratch_types=[
            pltpu.VMEM((_BPW * _D,), jnp.float32),
            pltpu.VMEM((_BPW * _D,), jnp.float32),
            pltpu.VMEM((_BPW,), jnp.float32),
        ],
    )
    return dot(ue, ie)
